# Initial kernel scaffold; baseline (speedup 1.0000x reference)
import jax
import jax.numpy as jnp
from jax.experimental import pallas as pl


def kernel(x, edge_index, params):
    raise NotImplementedError("write your pallas kernel here")


# speedup vs baseline: 7.3880x; 7.3880x over previous
import functools

import jax
import jax.numpy as jnp
from jax import lax
from jax.experimental import pallas as pl
from jax.experimental.pallas import tpu as pltpu
from jax.experimental.pallas import tpu_sc as plsc

B, N, M, H, E = 4, 10000, 128, 32, 160000
NC, NS = 2, 16
NW = NC * NS
EPW = E // NW
CH = 40
NCHUNK = EPW // CH
NPAD = 10240
RPT = NPAD // NS
DOUTS = [H, 2 * H, 4 * H, 4 * H, 4 * H]

_MESH = plsc.VectorSubcoreMesh(
    core_axis_name="c", subcore_axis_name="s", num_cores=NC, num_subcores=NS)
_SC_PARAMS = pltpu.CompilerParams(use_tc_tiling_on_sc=False)


def _sc_deg_body(ei_hbm, ones_hbm, zeros_hbm, out_hbm, slab, ones_v, acc):
    c = lax.axis_index("c")
    s = lax.axis_index("s")
    w = c * NS + s
    r0 = s * RPT
    pltpu.sync_copy(ones_hbm, ones_v)
    for b in range(B):
        pltpu.sync_copy(zeros_hbm.at[pl.ds(r0, RPT)], acc.at[pl.ds(r0, RPT)])
        plsc.subcore_barrier()
        pltpu.sync_copy(ei_hbm.at[b, w], slab)

        def body(j, carry):
            pltpu.sync_copy(ones_v, acc.at[slab.at[j, 1]], add=True)
            return carry

        lax.fori_loop(0, NCHUNK, body, 0)
        plsc.subcore_barrier()
        pltpu.sync_copy(acc.at[pl.ds(r0, RPT)],
                        out_hbm.at[b, c, pl.ds(r0, RPT)])


_sc_deg = pl.kernel(
    _sc_deg_body,
    out_type=jax.ShapeDtypeStruct((B, NC, NPAD, 16), jnp.float32),
    mesh=_MESH,
    scratch_types=[
        pltpu.VMEM((NCHUNK, 2, CH), jnp.int32),
        pltpu.VMEM((CH, 16), jnp.float32),
        pltpu.VMEM_SHARED((NPAD, 16), jnp.float32),
    ],
    compiler_params=_SC_PARAMS,
)


def _sc_agg_body(dout, g_hbm, ei_hbm, zeros_hbm, out_hbm, slab, rows, acc):
    c = lax.axis_index("c")
    s = lax.axis_index("s")
    w = c * NS + s
    r0 = s * RPT
    for b in range(B):
        pltpu.sync_copy(zeros_hbm.at[pl.ds(r0, RPT)], acc.at[pl.ds(r0, RPT)])
        plsc.subcore_barrier()
        pltpu.sync_copy(ei_hbm.at[b, w], slab)

        def body(j, carry):
            pltpu.sync_copy(g_hbm.at[slab.at[j, 0]], rows)
            pltpu.sync_copy(rows, acc.at[slab.at[j, 1]], add=True)
            return carry

        lax.fori_loop(0, NCHUNK, body, 0)
        plsc.subcore_barrier()
        pltpu.sync_copy(acc.at[pl.ds(r0, RPT)],
                        out_hbm.at[b, c, pl.ds(r0, RPT)])


@functools.cache
def _sc_agg(dout):
    return pl.kernel(
        functools.partial(_sc_agg_body, dout),
        out_type=jax.ShapeDtypeStruct((B, NC, NPAD, dout), jnp.float32),
        mesh=_MESH,
        scratch_types=[
            pltpu.VMEM((NCHUNK, 2, CH), jnp.int32),
            pltpu.VMEM((CH, dout), jnp.float32),
            pltpu.VMEM_SHARED((NPAD, dout), jnp.float32),
        ],
        compiler_params=_SC_PARAMS,
    )


RC = 5
CHR = N // RC


def _tc_dinv_body(deg_ref, out_ref):
    deg = deg_ref[0, 0, :, 0:1] + deg_ref[0, 1, :, 0:1] + 1.0
    out_ref[0] = lax.rsqrt(deg)


def _tc_dinv(deg):
    return pl.pallas_call(
        _tc_dinv_body,
        grid=(B,),
        in_specs=[pl.BlockSpec((1, NC, NPAD, 16), lambda b: (b, 0, 0, 0))],
        out_specs=pl.BlockSpec((1, NPAD, 1), lambda b: (b, 0, 0)),
        out_shape=jax.ShapeDtypeStruct((B, NPAD, 1), jnp.float32),
    )(deg)


def _tc_first_body(x_ref, dinv_ref, w_ref, g_ref):
    hw = jnp.dot(x_ref[0], w_ref[...], preferred_element_type=jnp.float32)
    g_ref[0] = dinv_ref[0] * hw


def _tc_first(xs, dinv, w1):
    return pl.pallas_call(
        _tc_first_body,
        grid=(B, RC),
        in_specs=[pl.BlockSpec((1, CHR, M), lambda b, r: (b, r, 0)),
                  pl.BlockSpec((1, CHR, 1), lambda b, r: (b, r, 0)),
                  pl.BlockSpec((M, H), lambda b, r: (0, 0))],
        out_specs=pl.BlockSpec((1, CHR, H), lambda b, r: (b, r, 0)),
        out_shape=jax.ShapeDtypeStruct((B, N, H), jnp.float32),
    )(xs, dinv, w1)


def _tc_act_body(s_ref, g_ref, dinv_ref, b_ref, act_ref, st_ref):
    tot = s_ref[0, 0] + s_ref[0, 1] + g_ref[0]
    pre = dinv_ref[0] * tot + b_ref[...]
    act = jnp.where(pre >= 0, pre, 0.01 * pre)
    act_ref[0] = act
    st_ref[0, 0, 0] = jnp.sum(act, axis=0)
    st_ref[0, 0, 1] = jnp.sum(act * act, axis=0)


def _tc_act(s, g, dinv, bi, dout):
    return pl.pallas_call(
        _tc_act_body,
        grid=(B, RC),
        in_specs=[pl.BlockSpec((1, NC, CHR, dout), lambda b, r: (b, 0, r, 0)),
                  pl.BlockSpec((1, CHR, dout), lambda b, r: (b, r, 0)),
                  pl.BlockSpec((1, CHR, 1), lambda b, r: (b, r, 0)),
                  pl.BlockSpec((1, dout), lambda b, r: (0, 0))],
        out_specs=[pl.BlockSpec((1, CHR, dout), lambda b, r: (b, r, 0)),
                   pl.BlockSpec((1, 1, 2, dout), lambda b, r: (b, r, 0, 0))],
        out_shape=[jax.ShapeDtypeStruct((B, N, dout), jnp.float32),
                   jax.ShapeDtypeStruct((B, RC, 2, dout), jnp.float32)],
    )(s, g, dinv, bi)


def _bn(act_ref, st_ref, bng_ref, bnb_ref):
    st = st_ref[0]
    mu = jnp.sum(st[:, 0, :], axis=0, keepdims=True) * (1.0 / N)
    sq = jnp.sum(st[:, 1, :], axis=0, keepdims=True) * (1.0 / N)
    var = sq - mu * mu
    return (act_ref[0] - mu) * lax.rsqrt(var + 1e-5) * bng_ref[...] + bnb_ref[...]


def _tc_mid_body(act_ref, st_ref, dinv_ref, bng_ref, bnb_ref, w_ref, out_ref):
    h = _bn(act_ref, st_ref, bng_ref, bnb_ref)
    hw = jnp.dot(h, w_ref[...], preferred_element_type=jnp.float32)
    out_ref[0] = dinv_ref[0] * hw


def _tc_mid(act, st, dinv, bng, bnb, wn, dout, dnext):
    return pl.pallas_call(
        _tc_mid_body,
        grid=(B, RC),
        in_specs=[pl.BlockSpec((1, CHR, dout), lambda b, r: (b, r, 0)),
                  pl.BlockSpec((1, RC, 2, dout), lambda b, r: (b, 0, 0, 0)),
                  pl.BlockSpec((1, CHR, 1), lambda b, r: (b, r, 0)),
                  pl.BlockSpec((1, dout), lambda b, r: (0, 0)),
                  pl.BlockSpec((1, dout), lambda b, r: (0, 0)),
                  pl.BlockSpec((dout, dnext), lambda b, r: (0, 0))],
        out_specs=pl.BlockSpec((1, CHR, dnext), lambda b, r: (b, r, 0)),
        out_shape=jax.ShapeDtypeStruct((B, N, dnext), jnp.float32),
    )(act, st, dinv, bng, bnb, wn)


def _tc_last_body(act_ref, st_ref, bng_ref, bnb_ref, lng_ref, lnb_ref,
                  out_ref):
    h = _bn(act_ref, st_ref, bng_ref, bnb_ref)
    mu = jnp.mean(h, axis=-1, keepdims=True)
    xc = h - mu
    var = jnp.mean(xc * xc, axis=-1, keepdims=True)
    out_ref[0] = xc * lax.rsqrt(var + 1e-5) * lng_ref[...] + lnb_ref[...]


def _tc_last(act, st, bng, bnb, lng, lnb, dout):
    return pl.pallas_call(
        _tc_last_body,
        grid=(B, RC),
        in_specs=[pl.BlockSpec((1, CHR, dout), lambda b, r: (b, r, 0)),
                  pl.BlockSpec((1, RC, 2, dout), lambda b, r: (b, 0, 0, 0)),
                  pl.BlockSpec((1, dout), lambda b, r: (0, 0)),
                  pl.BlockSpec((1, dout), lambda b, r: (0, 0)),
                  pl.BlockSpec((1, dout), lambda b, r: (0, 0)),
                  pl.BlockSpec((1, dout), lambda b, r: (0, 0))],
        out_specs=pl.BlockSpec((1, CHR, dout), lambda b, r: (b, r, 0)),
        out_shape=jax.ShapeDtypeStruct((B, N, dout), jnp.float32),
    )(act, st, bng, bnb, lng, lnb)


def kernel(x, edge_index, params):
    xs = jnp.squeeze(x, -1)
    src = edge_index[:, 0, :] + (jnp.arange(B, dtype=jnp.int32) * N)[:, None]
    dst = edge_index[:, 1, :]
    ei = jnp.stack([src.reshape(B, NW, NCHUNK, CH),
                    dst.reshape(B, NW, NCHUNK, CH)], axis=3)
    ones16 = jnp.ones((CH, 16), jnp.float32)
    z16 = jnp.zeros((NPAD, 16), jnp.float32)
    deg = _sc_deg(ei, ones16, z16)
    dinv = _tc_dinv(deg)

    p = params
    vec = lambda v: v.reshape(1, -1)
    g = _tc_first(xs, dinv, p["W1"])
    for i in range(1, 6):
        dout = DOUTS[i - 1]
        zer = jnp.zeros((NPAD, dout), jnp.float32)
        s = _sc_agg(dout)(g.reshape(B * N, dout), ei, zer)
        act, st = _tc_act(s, g, dinv, vec(p[f"b{i}"]), dout)
        if i < 5:
            g = _tc_mid(act, st, dinv, vec(p[f"bn{i}_g"]), vec(p[f"bn{i}_b"]),
                        p[f"W{i+1}"], dout, DOUTS[i])
        else:
            out = _tc_last(act, st, vec(p["bn5_g"]), vec(p["bn5_b"]),
                           vec(p["ln_g"]), vec(p["ln_b"]), dout)
    return out.reshape(B, N * DOUTS[4])
